# baseline (speedup 1.0000x reference)
import jax
import jax.numpy as jnp
from jax.experimental import pallas as pl
from jax.experimental.pallas import tpu as pltpu

_BN_EPS = 1e-5

_G0 = 4
_G1 = 2


def _softsign(z):
    return z / (1.0 + jnp.abs(z))


def _x_index_map(NB0, B):
    CUT = (B - 2 * _G0) // _G1

    def index_map(j):
        rd = jnp.where(j == NB0 - 2, NB0 - 1,
                       jnp.where(j == NB0 - 1, NB0 - 2, j))
        wr = jnp.where(j - NB0 - 1 >= CUT + (_G0 // _G1), NB0 - 1, NB0 - 2)
        return (jnp.where(j < NB0, rd, wr), 0, 0)

    return index_map


def _fused_kernel(qp_ref, wa_ref, ba_ref, wb_ref, bb_ref,
                  wc_ref, bc_ref, w_ref, qt_ref, pt_ref, x_ref,
                  out_ref, pooled_scr, scale_scr, spat_scr, xc_scr):
    j = pl.program_id(0)
    NB0 = pooled_scr.shape[0]
    Bsz = NB0 * _G0
    R, T = wa_ref.shape[0], wa_ref.shape[1]
    Nx = wb_ref.shape[1]
    Ny = wc_ref.shape[1]
    D = T + Nx + Ny
    S = x_ref.shape[1]
    CUT = (Bsz - 2 * _G0) // _G1

    @pl.when(j < NB0)
    def _pool():
        bi = jnp.where(j == NB0 - 2, NB0 - 1,
                       jnp.where(j == NB0 - 1, NB0 - 2, j))
        Xbf = x_ref[...].astype(jnp.bfloat16)
        ones_s = jnp.ones((1, S), jnp.bfloat16)
        ones_t = jnp.ones((1, T), jnp.float32)
        rows = []
        for g in range(_G0):
            pa = jnp.dot(ones_s, Xbf[g],
                         preferred_element_type=jnp.float32) * (1.0 / S)
            z = jax.lax.dot_general(
                Xbf[g], qp_ref[...], (((0,), (0,)), ((), ())),
                preferred_element_type=jnp.float32)
            pbc = jnp.dot(ones_t, z, preferred_element_type=jnp.float32)
            rows.append(jnp.concatenate([pa, pbc], axis=1))
        pooled_scr[bi] = jnp.concatenate(rows, axis=0)

        @pl.when(j < NB0 - 2)
        def _fill():
            xc_scr[j] = Xbf

    @pl.when(j == NB0)
    def _chain():
        pooled = pooled_scr[...].reshape(Bsz, D)
        off = jnp.zeros((1, T), jnp.float32)

        for r in range(R):
            pa = pooled[:, 0:T]
            pb = pooled[:, T:T + Nx]
            pc = pooled[:, T + Nx:D]
            av = _softsign(jax.lax.dot_general(
                pa, wa_ref[r], (((1,), (1,)), ((), ())),
                preferred_element_type=jnp.float32) + ba_ref[r])
            bv = _softsign(jax.lax.dot_general(
                pb, wb_ref[r], (((1,), (1,)), ((), ())),
                preferred_element_type=jnp.float32) + bb_ref[r])
            cv = _softsign(jax.lax.dot_general(
                pc, wc_ref[r], (((1,), (1,)), ((), ())),
                preferred_element_type=jnp.float32) + bc_ref[r])

            bbar = jnp.mean(bv, axis=1, keepdims=True)
            cbar = jnp.mean(cv, axis=1, keepdims=True)
            b2 = jnp.mean(bv * bv, axis=1, keepdims=True)
            c2 = jnp.mean(cv * cv, axis=1, keepdims=True)
            mu = jnp.mean(av * (bbar * cbar), axis=0, keepdims=True)
            m2 = jnp.mean((av * av) * (b2 * c2), axis=0, keepdims=True)
            var = jnp.maximum(m2 - mu * mu, 0.0)
            inv = jax.lax.rsqrt(var + _BN_EPS)

            wr = w_ref[r]
            scale_scr[:, r, :] = ((wr * inv) * av).astype(jnp.bfloat16)
            spat_scr[:, r, :] = (
                jnp.dot(bv.astype(jnp.bfloat16), qt_ref[...],
                        preferred_element_type=jnp.float32) *
                jnp.dot(cv.astype(jnp.bfloat16), pt_ref[...],
                        preferred_element_type=jnp.float32)
            ).astype(jnp.bfloat16)
            off = off + wr * (inv * mu)

            if r + 1 < R:
                pa_n = pa - inv * (av * (bbar * cbar) - mu)
                a1 = jnp.mean(inv * av, axis=1, keepdims=True)
                m1 = jnp.mean(inv * mu, axis=1, keepdims=True)
                pb_n = pb - (bv * (cbar * a1) - m1)
                pc_n = pc - (cv * (bbar * a1) - m1)
                pooled = jnp.concatenate([pa_n, pb_n, pc_n], axis=1)

        scale_scr[:, R, :] = jnp.broadcast_to(-off, (Bsz, T)).astype(jnp.bfloat16)
        spat_scr[:, R, :] = jnp.ones((Bsz, S), jnp.bfloat16)

    def _delta(bb):
        sc = scale_scr[bb]
        sp = spat_scr[bb]
        return jax.lax.dot_general(
            sp, sc, (((0,), (0,)), ((), ())),
            preferred_element_type=jnp.float32)

    oi = j - NB0 - 1

    @pl.when(jnp.logical_and(j > NB0, oi < CUT))
    def _apply_cached():
        gi = oi // (_G0 // _G1)
        o0 = (oi % (_G0 // _G1)) * _G1
        for g in range(_G1):
            Xg = xc_scr[gi, o0 + g]
            out_ref[g] = Xg.astype(jnp.float32) + _delta(oi * _G1 + g)

    @pl.when(oi >= CUT)
    def _apply_parked():
        o0 = (oi % (_G0 // _G1)) * _G1
        for g in range(_G1):
            out_ref[g] = x_ref[o0 + g] + _delta(oi * _G1 + g)


def _cp_forward(x, Wa, ba, Wb, bb, Wc, bc, w):
    B, T, Nx, Ny = x.shape
    S = Nx * Ny
    R = Wa.shape[0]
    R1 = R + 1
    D = T + Nx + Ny
    NB0 = B // _G0
    NB1 = B // _G1

    xt = jnp.transpose(x, (0, 2, 3, 1)).reshape(B, S, T)

    s_idx = jnp.arange(S, dtype=jnp.int32)
    Q = (s_idx[:, None] // Ny == jnp.arange(Nx, dtype=jnp.int32)[None, :]
         ).astype(jnp.float32)
    P = (s_idx[:, None] % Ny == jnp.arange(Ny, dtype=jnp.int32)[None, :]
         ).astype(jnp.float32)
    QP = jnp.concatenate([Q * (1.0 / (T * Ny)), P * (1.0 / (T * Nx))],
                         axis=1).astype(jnp.bfloat16)

    smem = pl.BlockSpec(memory_space=pltpu.MemorySpace.SMEM)

    out_t = pl.pallas_call(
        _fused_kernel,
        out_shape=jax.ShapeDtypeStruct((B, S, T), x.dtype),
        grid=(NB0 + 1 + NB1,),
        in_specs=[
            pl.BlockSpec((S, Nx + Ny), lambda j: (0, 0)),
            pl.BlockSpec((R, T, T), lambda j: (0, 0, 0)),
            pl.BlockSpec((R, 1, T), lambda j: (0, 0, 0)),
            pl.BlockSpec((R, Nx, Nx), lambda j: (0, 0, 0)),
            pl.BlockSpec((R, 1, Nx), lambda j: (0, 0, 0)),
            pl.BlockSpec((R, Ny, Ny), lambda j: (0, 0, 0)),
            pl.BlockSpec((R, 1, Ny), lambda j: (0, 0, 0)),
            smem,
            pl.BlockSpec((Nx, S), lambda j: (0, 0)),
            pl.BlockSpec((Ny, S), lambda j: (0, 0)),
            pl.BlockSpec((_G0, S, T), _x_index_map(NB0, B)),
        ],
        out_specs=pl.BlockSpec((_G1, S, T),
                               lambda j: (jnp.maximum(j - (NB0 + 1), 0), 0, 0)),
        scratch_shapes=[
            pltpu.VMEM((NB0, _G0, D), jnp.float32),
            pltpu.VMEM((B, R1, T), jnp.bfloat16),
            pltpu.VMEM((B, R1, S), jnp.bfloat16),
            pltpu.VMEM((NB0 - 2, _G0, S, T), jnp.bfloat16),
        ],
        compiler_params=pltpu.CompilerParams(
            dimension_semantics=("arbitrary",)),
    )(QP, Wa, ba, Wb, bb, Wc, bc, w,
      Q.T.astype(jnp.bfloat16), P.T.astype(jnp.bfloat16), xt)

    return out_t.reshape(B, Nx, Ny, T).transpose(0, 3, 1, 2)


def kernel(x, Wa, ba, Wb, bb, Wc, bc, w):
    return _cp_forward(x, Wa, ba, Wb, bb, Wc, bc, w)


# speedup vs baseline: 1.0242x; 1.0242x over previous
import jax
import jax.numpy as jnp
from jax.experimental import pallas as pl
from jax.experimental.pallas import tpu as pltpu

_BN_EPS = 1e-5


def _softsign(z):
    return z / (1.0 + jnp.abs(z))


def _fused_kernel(qp_ref, wa_ref, ba_ref, wb_ref, bb_ref,
                  wc_ref, bc_ref, w_ref, qt_ref, pt_ref, x_ref,
                  out_ref, pooled_scr, scale_scr, spat_scr, xc_scr):
    p = pl.program_id(0)
    i = pl.program_id(1)
    Bsz = pooled_scr.shape[0] * pooled_scr.shape[1]
    R, T = wa_ref.shape[0], wa_ref.shape[1]
    Nx = wb_ref.shape[1]
    Ny = wc_ref.shape[1]
    D = T + Nx + Ny
    G = x_ref.shape[0]
    S = x_ref.shape[1]

    @pl.when(p == 0)
    def _pool():
        Xbf = x_ref[...].astype(jnp.bfloat16)
        xc_scr[i] = Xbf
        ones_s = jnp.ones((1, S), jnp.bfloat16)
        ones_t = jnp.ones((1, T), jnp.float32)
        rows = []
        for g in range(G):
            pa = jnp.dot(ones_s, Xbf[g],
                         preferred_element_type=jnp.float32) * (1.0 / S)
            z = jax.lax.dot_general(
                Xbf[g], qp_ref[...], (((0,), (0,)), ((), ())),
                preferred_element_type=jnp.float32)
            pbc = jnp.dot(ones_t, z, preferred_element_type=jnp.float32)
            rows.append(jnp.concatenate([pa, pbc], axis=1))
        pooled_scr[i] = jnp.concatenate(rows, axis=0)

    @pl.when(jnp.logical_and(p == 1, i == 0))
    def _chain():
        pooled = pooled_scr[...].reshape(Bsz, D)
        off = jnp.zeros((1, T), jnp.float32)

        for r in range(R):
            pa = pooled[:, 0:T]
            pb = pooled[:, T:T + Nx]
            pc = pooled[:, T + Nx:D]
            av = _softsign(jax.lax.dot_general(
                pa, wa_ref[r], (((1,), (1,)), ((), ())),
                preferred_element_type=jnp.float32) + ba_ref[r])
            bv = _softsign(jax.lax.dot_general(
                pb, wb_ref[r], (((1,), (1,)), ((), ())),
                preferred_element_type=jnp.float32) + bb_ref[r])
            cv = _softsign(jax.lax.dot_general(
                pc, wc_ref[r], (((1,), (1,)), ((), ())),
                preferred_element_type=jnp.float32) + bc_ref[r])

            bbar = jnp.mean(bv, axis=1, keepdims=True)
            cbar = jnp.mean(cv, axis=1, keepdims=True)
            b2 = jnp.mean(bv * bv, axis=1, keepdims=True)
            c2 = jnp.mean(cv * cv, axis=1, keepdims=True)
            mu = jnp.mean(av * (bbar * cbar), axis=0, keepdims=True)
            m2 = jnp.mean((av * av) * (b2 * c2), axis=0, keepdims=True)
            var = jnp.maximum(m2 - mu * mu, 0.0)
            inv = jax.lax.rsqrt(var + _BN_EPS)

            wr = w_ref[r]
            scale_scr[:, r, :] = ((wr * inv) * av).astype(jnp.bfloat16)
            spat_scr[:, r, :] = (
                jnp.dot(bv.astype(jnp.bfloat16), qt_ref[...],
                        preferred_element_type=jnp.float32) *
                jnp.dot(cv.astype(jnp.bfloat16), pt_ref[...],
                        preferred_element_type=jnp.float32)
            ).astype(jnp.bfloat16)
            off = off + wr * (inv * mu)

            if r + 1 < R:
                pa_n = pa - inv * (av * (bbar * cbar) - mu)
                a1 = jnp.mean(inv * av, axis=1, keepdims=True)
                m1 = jnp.mean(inv * mu, axis=1, keepdims=True)
                pb_n = pb - (bv * (cbar * a1) - m1)
                pc_n = pc - (cv * (bbar * a1) - m1)
                pooled = jnp.concatenate([pa_n, pb_n, pc_n], axis=1)

        scale_scr[:, R, :] = jnp.broadcast_to(-off, (Bsz, T)).astype(jnp.bfloat16)
        spat_scr[:, R, :] = jnp.ones((Bsz, S), jnp.bfloat16)

    @pl.when(p == 1)
    def _apply():
        for g in range(G):
            sc = scale_scr[i * G + g]
            sp = spat_scr[i * G + g]
            delta = jax.lax.dot_general(
                sp, sc, (((0,), (0,)), ((), ())),
                preferred_element_type=jnp.float32)
            out_ref[g] = xc_scr[i, g].astype(jnp.float32) + delta


def _cp_forward(x, Wa, ba, Wb, bb, Wc, bc, w):
    B, T, Nx, Ny = x.shape
    S = Nx * Ny
    R = Wa.shape[0]
    R1 = R + 1
    D = T + Nx + Ny

    xt = jnp.transpose(x, (0, 2, 3, 1)).reshape(B, S, T)

    s_idx = jnp.arange(S, dtype=jnp.int32)
    Q = (s_idx[:, None] // Ny == jnp.arange(Nx, dtype=jnp.int32)[None, :]
         ).astype(jnp.float32)
    P = (s_idx[:, None] % Ny == jnp.arange(Ny, dtype=jnp.int32)[None, :]
         ).astype(jnp.float32)
    QP = jnp.concatenate([Q * (1.0 / (T * Ny)), P * (1.0 / (T * Nx))],
                         axis=1).astype(jnp.bfloat16)

    smem = pl.BlockSpec(memory_space=pltpu.MemorySpace.SMEM)
    G = 2
    NB = B // G

    out_t = pl.pallas_call(
        _fused_kernel,
        out_shape=jax.ShapeDtypeStruct((B, S, T), x.dtype),
        grid=(2, B // G),
        in_specs=[
            pl.BlockSpec((S, Nx + Ny), lambda p, i: (0, 0)),
            pl.BlockSpec((R, T, T), lambda p, i: (0, 0, 0)),
            pl.BlockSpec((R, 1, T), lambda p, i: (0, 0, 0)),
            pl.BlockSpec((R, Nx, Nx), lambda p, i: (0, 0, 0)),
            pl.BlockSpec((R, 1, Nx), lambda p, i: (0, 0, 0)),
            pl.BlockSpec((R, Ny, Ny), lambda p, i: (0, 0, 0)),
            pl.BlockSpec((R, 1, Ny), lambda p, i: (0, 0, 0)),
            smem,
            pl.BlockSpec((Nx, S), lambda p, i: (0, 0)),
            pl.BlockSpec((Ny, S), lambda p, i: (0, 0)),
            pl.BlockSpec((G, S, T),
                         lambda p, i: ((1 - p) * i + p * (NB - 1), 0, 0)),
        ],
        out_specs=pl.BlockSpec((G, S, T), lambda p, i: (p * i, 0, 0)),
        scratch_shapes=[
            pltpu.VMEM((NB, G, D), jnp.float32),
            pltpu.VMEM((B, R1, T), jnp.bfloat16),
            pltpu.VMEM((B, R1, S), jnp.bfloat16),
            pltpu.VMEM((NB, G, S, T), jnp.bfloat16),
        ],
        compiler_params=pltpu.CompilerParams(
            dimension_semantics=("arbitrary", "arbitrary")),
    )(QP, Wa, ba, Wb, bb, Wc, bc, w,
      Q.T.astype(jnp.bfloat16), P.T.astype(jnp.bfloat16), xt)

    return out_t.reshape(B, Nx, Ny, T).transpose(0, 3, 1, 2)


def kernel(x, Wa, ba, Wb, bb, Wc, bc, w):
    return _cp_forward(x, Wa, ba, Wb, bb, Wc, bc, w)
